# baseline (speedup 1.0000x reference)
import functools

import jax
import jax.numpy as jnp
from jax.experimental import pallas as pl
from jax.experimental.pallas import tpu as pltpu

_N_EXPERTS = 64
_TOP_K = 8
_TILE = 512


def _gate_kernel(x_ref, w_ref, idx_ref, wgt_ref):
    x = x_ref[...]
    w = w_ref[...]
    logits = jax.lax.dot_general(
        x, w, (((1,), (1,)), ((), ())), preferred_element_type=jnp.float32
    )
    t = logits.shape[0]
    lane = jax.lax.broadcasted_iota(jnp.int32, (t, _N_EXPERTS), 1)
    scores = logits
    vals = []
    idxs = []
    for _ in range(_TOP_K):
        m = jnp.max(scores, axis=1, keepdims=True)
        idx = jnp.min(jnp.where(scores == m, lane, _N_EXPERTS), axis=1, keepdims=True)
        vals.append(m)
        idxs.append(idx)
        scores = jnp.where(lane == idx, -jnp.inf, scores)
    top_vals = jnp.concatenate(vals, axis=1)
    top_idx = jnp.concatenate(idxs, axis=1)
    e = jnp.exp(top_vals - top_vals[:, :1])
    wgt_ref[...] = e / jnp.sum(e, axis=1, keepdims=True)
    idx_ref[...] = top_idx


@functools.partial(jax.jit, static_argnums=())
def kernel(hidden_states, weight):
    bsz, seq, h = hidden_states.shape
    tokens = bsz * seq
    x = hidden_states.reshape(tokens, h).astype(jnp.float32)
    w = weight.astype(jnp.float32)
    grid = (tokens // _TILE,)
    idx, wgt = pl.pallas_call(
        _gate_kernel,
        grid=grid,
        in_specs=[
            pl.BlockSpec((_TILE, h), lambda i: (i, 0)),
            pl.BlockSpec((_N_EXPERTS, h), lambda i: (0, 0)),
        ],
        out_specs=[
            pl.BlockSpec((_TILE, _TOP_K), lambda i: (i, 0)),
            pl.BlockSpec((_TILE, _TOP_K), lambda i: (i, 0)),
        ],
        out_shape=[
            jax.ShapeDtypeStruct((tokens, _TOP_K), jnp.int32),
            jax.ShapeDtypeStruct((tokens, _TOP_K), jnp.float32),
        ],
        compiler_params=pltpu.CompilerParams(
            dimension_semantics=("parallel",)
        ),
    )(x, w)
    return idx, wgt


# speedup vs baseline: 1.1527x; 1.1527x over previous
import functools

import jax
import jax.numpy as jnp
from jax.experimental import pallas as pl
from jax.experimental.pallas import tpu as pltpu

_N_EXPERTS = 64
_TOP_K = 8
_TILE = 512


def _gate_kernel(x_ref, w_ref, idx_ref, wgt_ref):
    x = x_ref[...]
    w = w_ref[...]
    logits = jax.lax.dot_general(
        x, w, (((1,), (1,)), ((), ())), preferred_element_type=jnp.float32
    )
    t = logits.shape[0]
    rev_lane = jnp.int32(_N_EXPERTS - 1) - jax.lax.broadcasted_iota(
        jnp.int32, (t, _N_EXPERTS), 1
    )
    b = jax.lax.bitcast_convert_type(logits, jnp.int32)
    sortable = jnp.where(b < 0, b ^ jnp.int32(0x7FFFFFFF), b)
    key = (sortable & jnp.int32(-_N_EXPERTS)) | rev_lane
    neg_inf_key = jnp.iinfo(jnp.int32).min
    keys = []
    for _ in range(_TOP_K):
        m = jnp.max(key, axis=1, keepdims=True)
        keys.append(m)
        key = jnp.where(key == m, neg_inf_key, key)
    top = jnp.concatenate(keys, axis=1)
    top_idx = jnp.int32(_N_EXPERTS - 1) - (top & jnp.int32(_N_EXPERTS - 1))
    vb = top & jnp.int32(-_N_EXPERTS)
    vb = jnp.where(vb < 0, vb ^ jnp.int32(0x7FFFFFFF), vb)
    top_vals = jax.lax.bitcast_convert_type(vb, jnp.float32)
    e = jnp.exp(top_vals - top_vals[:, :1])
    wgt_ref[...] = e / jnp.sum(e, axis=1, keepdims=True)
    idx_ref[...] = top_idx


@functools.partial(jax.jit, static_argnums=())
def kernel(hidden_states, weight):
    bsz, seq, h = hidden_states.shape
    tokens = bsz * seq
    x = hidden_states.reshape(tokens, h).astype(jnp.float32)
    w = weight.astype(jnp.float32)
    grid = (tokens // _TILE,)
    idx, wgt = pl.pallas_call(
        _gate_kernel,
        grid=grid,
        in_specs=[
            pl.BlockSpec((_TILE, h), lambda i: (i, 0)),
            pl.BlockSpec((_N_EXPERTS, h), lambda i: (0, 0)),
        ],
        out_specs=[
            pl.BlockSpec((_TILE, _TOP_K), lambda i: (i, 0)),
            pl.BlockSpec((_TILE, _TOP_K), lambda i: (i, 0)),
        ],
        out_shape=[
            jax.ShapeDtypeStruct((tokens, _TOP_K), jnp.int32),
            jax.ShapeDtypeStruct((tokens, _TOP_K), jnp.float32),
        ],
        compiler_params=pltpu.CompilerParams(
            dimension_semantics=("parallel",)
        ),
    )(x, w)
    return idx, wgt


# speedup vs baseline: 1.3852x; 1.2017x over previous
import functools

import jax
import jax.numpy as jnp
from jax.experimental import pallas as pl
from jax.experimental.pallas import tpu as pltpu

_N_EXPERTS = 64
_TOP_K = 8
_TILE = 512


def _gate_kernel(x_ref, w_ref, idx_ref, wgt_ref):
    x = x_ref[...]
    w = w_ref[...]
    logits = jax.lax.dot_general(
        x, w, (((1,), (1,)), ((), ())), preferred_element_type=jnp.float32
    )
    t = logits.shape[0]
    rev_lane = jnp.int32(_N_EXPERTS - 1) - jax.lax.broadcasted_iota(
        jnp.int32, (t, _N_EXPERTS), 1
    )
    rm = jnp.max(logits, axis=1, keepdims=True)
    p = jnp.exp(logits - rm)
    b = jax.lax.bitcast_convert_type(p, jnp.int32)
    key = jax.lax.bitcast_convert_type(
        (b & jnp.int32(-_N_EXPERTS)) | rev_lane, jnp.float32
    )
    keys = []
    for _ in range(_TOP_K):
        m = jnp.max(key, axis=1, keepdims=True)
        keys.append(m)
        key = jnp.where(key == m, -1.0, key)
    top = jax.lax.bitcast_convert_type(
        jnp.concatenate(keys, axis=1), jnp.int32
    )
    idx_ref[...] = jnp.int32(_N_EXPERTS - 1) - (top & jnp.int32(_N_EXPERTS - 1))
    e = jax.lax.bitcast_convert_type(top & jnp.int32(-_N_EXPERTS), jnp.float32)
    wgt_ref[...] = e / jnp.sum(e, axis=1, keepdims=True)


@functools.partial(jax.jit, static_argnums=())
def kernel(hidden_states, weight):
    bsz, seq, h = hidden_states.shape
    tokens = bsz * seq
    x = hidden_states.reshape(tokens, h).astype(jnp.float32)
    w = weight.astype(jnp.float32)
    grid = (tokens // _TILE,)
    idx, wgt = pl.pallas_call(
        _gate_kernel,
        grid=grid,
        in_specs=[
            pl.BlockSpec((_TILE, h), lambda i: (i, 0)),
            pl.BlockSpec((_N_EXPERTS, h), lambda i: (0, 0)),
        ],
        out_specs=[
            pl.BlockSpec((_TILE, _TOP_K), lambda i: (i, 0)),
            pl.BlockSpec((_TILE, _TOP_K), lambda i: (i, 0)),
        ],
        out_shape=[
            jax.ShapeDtypeStruct((tokens, _TOP_K), jnp.int32),
            jax.ShapeDtypeStruct((tokens, _TOP_K), jnp.float32),
        ],
        compiler_params=pltpu.CompilerParams(
            dimension_semantics=("parallel",)
        ),
    )(x, w)
    return idx, wgt
